# baseline (speedup 1.0000x reference)
import jax
import jax.numpy as jnp
from jax.experimental import pallas as pl
from jax.experimental.pallas import tpu as pltpu

N = 2048
T = 128
NT = N // T

_F32 = jnp.float32
_BF16 = jnp.bfloat16


def _emd_body(x1a, x1b, x1c, x2a, x2b, x2c, out_ref,
              d2_ref, d_ref, e0_ref, e1_ref, rc0_ref, rc1_ref, ab_ref):
    b1 = x2a[0]
    b2 = x2b[0]
    b3 = x2c[0]

    ones_col = jnp.ones((N, 1), dtype=_BF16)

    def build(t, carry):
        rs = pl.ds(t * T, T)
        p1 = x1a[0, rs, :]
        p2 = x1b[0, rs, :]
        p3 = x1c[0, rs, :]
        d2_t = (p1 - b1) ** 2 + (p2 - b2) ** 2 + (p3 - b3) ** 2
        d2_ref[rs, :] = d2_t
        d_ref[rs, :] = jnp.sqrt(jnp.maximum(d2_t, 1e-12)).astype(_BF16)
        e_t = jnp.exp((-(4.0 ** 8)) * d2_t).astype(_BF16)
        e0_ref[rs, :] = e_t
        rc0_ref[:, rs] = jnp.transpose(
            jax.lax.dot(e_t, ones_col, preferred_element_type=_F32))
        return carry

    jax.lax.fori_loop(0, NT, build, 0)

    satl = jnp.ones((1, N), dtype=_F32)
    satr = jnp.ones((1, N), dtype=_F32)
    cost = jnp.zeros((1, N), dtype=_F32)
    s = rc0_ref[:, :]

    for idx in range(10):
        j = 8 - idx
        cur = e0_ref if idx % 2 == 0 else e1_ref
        nxt = e1_ref if idx % 2 == 0 else e0_ref
        has_next = idx < 9
        level_next = -(4.0 ** (j - 1))

        a = satl / (s + 1e-9)
        ab_ref[:, :] = a.astype(_BF16)

        def pass_l1(t, cs):
            rs = pl.ds(t * T, T)
            if has_next:
                nxt[rs, :] = jnp.exp(level_next * d2_ref[rs, :]).astype(_BF16)
            return cs + jax.lax.dot(
                ab_ref[:, rs], cur[rs, :], preferred_element_type=_F32
            )

        cs = jax.lax.fori_loop(0, NT, pass_l1,
                               jnp.zeros((1, N), dtype=_F32))

        ssr = satr * cs
        r = jnp.minimum(satr / (ssr + 1e-9), 1.0)
        u = satr * r
        satr = jnp.maximum(satr - ssr * r, 0.0)
        u_b = u.astype(_BF16)
        satr_b = satr.astype(_BF16)

        def pass_l2(t, cost_c):
            rs = pl.ds(t * T, T)
            e_t = cur[rs, :]
            q_t = e_t * u_b
            r_t = q_t * d_ref[rs, :]
            rc0_ref[:, rs] = jnp.transpose(
                jax.lax.dot(q_t, ones_col, preferred_element_type=_F32))
            if has_next:
                p_t = nxt[rs, :] * satr_b
                rc1_ref[:, rs] = jnp.transpose(
                    jax.lax.dot(p_t, ones_col, preferred_element_type=_F32))
            return cost_c + jax.lax.dot(
                ab_ref[:, rs], r_t, preferred_element_type=_F32
            )

        cost = jax.lax.fori_loop(0, NT, pass_l2, cost)

        eu = rc0_ref[:, :]
        satl = jnp.maximum(satl - a * eu, 0.0)
        if has_next:
            s = rc1_ref[:, :]

    s0 = jnp.sum(satr) + 1e-9
    lsum = jnp.sum(satl)
    ss = satr * (lsum / s0) + 1e-9
    r = jnp.minimum(satr / ss, 1.0)
    u_b = (satr * r).astype(_BF16)
    ab_ref[:, :] = (satl * (1.0 / s0)).astype(_BF16)

    def pass_final(t, cost_c):
        rs = pl.ds(t * T, T)
        r_t = d_ref[rs, :] * u_b
        return cost_c + jax.lax.dot(
            ab_ref[:, rs], r_t, preferred_element_type=_F32
        )

    cost = jax.lax.fori_loop(0, NT, pass_final, cost)

    out_ref[0] = jnp.sum(cost, axis=1, keepdims=True)


def kernel(input1, input2):
    B = input1.shape[0]
    x2t = jnp.transpose(input2, (0, 2, 1))
    ins = (
        input1[:, :, 0:1],
        input1[:, :, 1:2],
        input1[:, :, 2:3],
        x2t[:, 0:1, :],
        x2t[:, 1:2, :],
        x2t[:, 2:3, :],
    )
    col_spec = pl.BlockSpec((1, N, 1), lambda b: (b, 0, 0))
    row_spec = pl.BlockSpec((1, 1, N), lambda b: (b, 0, 0))
    out = pl.pallas_call(
        _emd_body,
        grid=(B,),
        in_specs=[col_spec, col_spec, col_spec, row_spec, row_spec, row_spec],
        out_specs=pl.BlockSpec((1, 1, 1), lambda b: (b, 0, 0)),
        out_shape=jax.ShapeDtypeStruct((B, 1, 1), jnp.float32),
        scratch_shapes=[
            pltpu.VMEM((N, N), _F32),
            pltpu.VMEM((N, N), _BF16),
            pltpu.VMEM((N, N), _BF16),
            pltpu.VMEM((N, N), _BF16),
            pltpu.VMEM((1, N), _F32),
            pltpu.VMEM((1, N), _F32),
            pltpu.VMEM((1, N), _BF16),
        ],
    )(*ins)
    return out[:, 0, 0]


# speedup vs baseline: 1.3536x; 1.3536x over previous
import jax
import jax.numpy as jnp
from jax.experimental import pallas as pl
from jax.experimental.pallas import tpu as pltpu

N = 2048
T = 256
NT = N // T

_F32 = jnp.float32
_BF16 = jnp.bfloat16


def _emd_body(x1a, x1b, x1c, x2a, x2b, x2c, out_ref,
              d2_ref, d_ref, e0_ref, e1_ref,
              ab0_ref, ab1_ref, af0_ref, af1_ref, sl0_ref, sl1_ref):
    b1 = x2a[0]
    b2 = x2b[0]
    b3 = x2c[0]

    ones_col = jnp.ones((N, 1), dtype=_BF16)

    for t in range(NT):
        rs = pl.ds(t * T, T)
        p1 = x1a[0, rs, :]
        p2 = x1b[0, rs, :]
        p3 = x1c[0, rs, :]
        d2_t = (p1 - b1) ** 2 + (p2 - b2) ** 2 + (p3 - b3) ** 2
        d2_ref[rs, :] = d2_t
        d_ref[rs, :] = jnp.sqrt(jnp.maximum(d2_t, 1e-12)).astype(_BF16)
        e_t = jnp.exp((-(4.0 ** 8)) * d2_t).astype(_BF16)
        e0_ref[rs, :] = e_t
        s_row = jnp.transpose(
            jax.lax.dot(e_t, ones_col, preferred_element_type=_F32))
        a_row = 1.0 / (s_row + 1e-9)
        af0_ref[:, rs] = a_row
        ab0_ref[:, rs] = a_row.astype(_BF16)
        sl0_ref[:, rs] = jnp.ones((1, T), dtype=_F32)

    def pass_cs0(t, cs):
        rs = pl.ds(t * T, T)
        return cs + jax.lax.dot(
            ab0_ref[:, rs], e0_ref[rs, :], preferred_element_type=_F32
        )

    cs = jax.lax.fori_loop(0, NT, pass_cs0,
                           jnp.zeros((1, N), dtype=_F32), unroll=2)

    satr = jnp.ones((1, N), dtype=_F32)
    cost = jnp.zeros((1, N), dtype=_F32)

    for idx in range(10):
        j = 8 - idx
        p = idx % 2
        cur = e0_ref if p == 0 else e1_ref
        nxt = e1_ref if p == 0 else e0_ref
        ab_o = ab0_ref if p == 0 else ab1_ref
        ab_n = ab1_ref if p == 0 else ab0_ref
        af_o = af0_ref if p == 0 else af1_ref
        af_n = af1_ref if p == 0 else af0_ref
        sl_o = sl0_ref if p == 0 else sl1_ref
        sl_n = sl1_ref if p == 0 else sl0_ref
        has_next = idx < 9
        level_next = -(4.0 ** (j - 1))

        ssr = satr * cs
        r = jnp.minimum(satr / (ssr + 1e-9), 1.0)
        u = satr * r
        satr = jnp.maximum(satr - ssr * r, 0.0)
        u_b = u.astype(_BF16)
        satr_b = satr.astype(_BF16)

        def fused(t, carry):
            cost_c, cs_c = carry
            rs = pl.ds(t * T, T)
            e_t = cur[rs, :]
            q_t = e_t * u_b
            r_t = q_t * d_ref[rs, :]
            eu_row = jnp.transpose(
                jax.lax.dot(q_t, ones_col, preferred_element_type=_F32))
            cost_c = cost_c + jax.lax.dot(
                ab_o[:, rs], r_t, preferred_element_type=_F32
            )
            satl_t = jnp.maximum(sl_o[:, rs] - af_o[:, rs] * eu_row, 0.0)
            sl_n[:, rs] = satl_t
            if has_next:
                e2_t = jnp.exp(level_next * d2_ref[rs, :]).astype(_BF16)
                nxt[rs, :] = e2_t
                p_t = e2_t * satr_b
                s2_row = jnp.transpose(
                    jax.lax.dot(p_t, ones_col, preferred_element_type=_F32))
                a_t = satl_t / (s2_row + 1e-9)
                af_n[:, rs] = a_t
                a_tb = a_t.astype(_BF16)
                ab_n[:, rs] = a_tb
                cs_c = cs_c + jax.lax.dot(
                    a_tb, e2_t, preferred_element_type=_F32
                )
            return cost_c, cs_c

        cost, cs = jax.lax.fori_loop(
            0, NT, fused,
            (cost, jnp.zeros((1, N), dtype=_F32)), unroll=2)

    satl = sl0_ref[:, :]
    s0 = jnp.sum(satr) + 1e-9
    lsum = jnp.sum(satl)
    ss = satr * (lsum / s0) + 1e-9
    r = jnp.minimum(satr / ss, 1.0)
    u_b = (satr * r).astype(_BF16)
    ab0_ref[:, :] = (satl * (1.0 / s0)).astype(_BF16)

    def pass_final(t, cost_c):
        rs = pl.ds(t * T, T)
        r_t = d_ref[rs, :] * u_b
        return cost_c + jax.lax.dot(
            ab0_ref[:, rs], r_t, preferred_element_type=_F32
        )

    cost = jax.lax.fori_loop(0, NT, pass_final, cost, unroll=2)

    out_ref[0] = jnp.sum(cost, axis=1, keepdims=True)


def kernel(input1, input2):
    B = input1.shape[0]
    x2t = jnp.transpose(input2, (0, 2, 1))
    ins = (
        input1[:, :, 0:1],
        input1[:, :, 1:2],
        input1[:, :, 2:3],
        x2t[:, 0:1, :],
        x2t[:, 1:2, :],
        x2t[:, 2:3, :],
    )
    col_spec = pl.BlockSpec((1, N, 1), lambda b: (b, 0, 0))
    row_spec = pl.BlockSpec((1, 1, N), lambda b: (b, 0, 0))
    out = pl.pallas_call(
        _emd_body,
        grid=(B,),
        in_specs=[col_spec, col_spec, col_spec, row_spec, row_spec, row_spec],
        out_specs=pl.BlockSpec((1, 1, 1), lambda b: (b, 0, 0)),
        out_shape=jax.ShapeDtypeStruct((B, 1, 1), jnp.float32),
        scratch_shapes=[
            pltpu.VMEM((N, N), _F32),
            pltpu.VMEM((N, N), _BF16),
            pltpu.VMEM((N, N), _BF16),
            pltpu.VMEM((N, N), _BF16),
            pltpu.VMEM((1, N), _BF16),
            pltpu.VMEM((1, N), _BF16),
            pltpu.VMEM((1, N), _F32),
            pltpu.VMEM((1, N), _F32),
            pltpu.VMEM((1, N), _F32),
            pltpu.VMEM((1, N), _F32),
        ],
    )(*ins)
    return out[:, 0, 0]


# speedup vs baseline: 1.4940x; 1.1037x over previous
import jax
import jax.numpy as jnp
from jax.experimental import pallas as pl
from jax.experimental.pallas import tpu as pltpu

N = 2048
T = 256
NT = N // T

_F32 = jnp.float32
_BF16 = jnp.bfloat16


def _emd_body(x1a, x1b, x1c, x2a, x2b, x2c, out_ref,
              d2_ref, d_ref, e0_ref, e1_ref, rc0_ref, rc1_ref, ab_ref):
    b1 = x2a[0]
    b2 = x2b[0]
    b3 = x2c[0]

    ones_col = jnp.ones((N, 1), dtype=_BF16)

    for t in range(NT):
        rs = pl.ds(t * T, T)
        p1 = x1a[0, rs, :]
        p2 = x1b[0, rs, :]
        p3 = x1c[0, rs, :]
        d2_t = (p1 - b1) ** 2 + (p2 - b2) ** 2 + (p3 - b3) ** 2
        d2_ref[rs, :] = d2_t
        d_ref[rs, :] = jnp.sqrt(jnp.maximum(d2_t, 1e-12)).astype(_BF16)
        e_t = jnp.exp((-(4.0 ** 8)) * d2_t).astype(_BF16)
        e0_ref[rs, :] = e_t
        rc0_ref[:, rs] = jnp.transpose(
            jax.lax.dot(e_t, ones_col, preferred_element_type=_F32))

    satl = jnp.ones((1, N), dtype=_F32)
    satr = jnp.ones((1, N), dtype=_F32)
    cost = jnp.zeros((1, N), dtype=_F32)
    s = rc0_ref[:, :]

    for idx in range(10):
        j = 8 - idx
        cur = e0_ref if idx % 2 == 0 else e1_ref
        nxt = e1_ref if idx % 2 == 0 else e0_ref
        has_next = idx < 9
        level_next = -(4.0 ** (j - 1))

        a = satl / (s + 1e-9)
        ab_ref[:, :] = a.astype(_BF16)

        def pass_l1(t, cs):
            rs = pl.ds(t * T, T)
            return cs + jax.lax.dot(
                ab_ref[:, rs], cur[rs, :], preferred_element_type=_F32
            )

        cs = jax.lax.fori_loop(0, NT, pass_l1,
                               jnp.zeros((1, N), dtype=_F32), unroll=2)

        ssr = satr * cs
        r = jnp.minimum(satr / (ssr + 1e-9), 1.0)
        u = satr * r
        satr = jnp.maximum(satr - ssr * r, 0.0)
        u_b = u.astype(_BF16)
        satr_b = satr.astype(_BF16)

        def pass_l2(t, cost_c):
            rs = pl.ds(t * T, T)
            e_t = cur[rs, :]
            q_t = e_t * u_b
            r_t = q_t * d_ref[rs, :]
            rc0_ref[:, rs] = jnp.transpose(
                jax.lax.dot(q_t, ones_col, preferred_element_type=_F32))
            if has_next:
                e2_t = jnp.exp(level_next * d2_ref[rs, :]).astype(_BF16)
                nxt[rs, :] = e2_t
                p_t = e2_t * satr_b
                rc1_ref[:, rs] = jnp.transpose(
                    jax.lax.dot(p_t, ones_col, preferred_element_type=_F32))
            return cost_c + jax.lax.dot(
                ab_ref[:, rs], r_t, preferred_element_type=_F32
            )

        cost = jax.lax.fori_loop(0, NT, pass_l2, cost, unroll=2)

        eu = rc0_ref[:, :]
        satl = jnp.maximum(satl - a * eu, 0.0)
        if has_next:
            s = rc1_ref[:, :]

    s0 = jnp.sum(satr) + 1e-9
    lsum = jnp.sum(satl)
    ss = satr * (lsum / s0) + 1e-9
    r = jnp.minimum(satr / ss, 1.0)
    u_b = (satr * r).astype(_BF16)
    ab_ref[:, :] = (satl * (1.0 / s0)).astype(_BF16)

    for t in range(NT):
        rs = pl.ds(t * T, T)
        r_t = d_ref[rs, :] * u_b
        cost = cost + jax.lax.dot(
            ab_ref[:, rs], r_t, preferred_element_type=_F32
        )

    out_ref[0] = jnp.sum(cost, axis=1, keepdims=True)


def kernel(input1, input2):
    B = input1.shape[0]
    x2t = jnp.transpose(input2, (0, 2, 1))
    ins = (
        input1[:, :, 0:1],
        input1[:, :, 1:2],
        input1[:, :, 2:3],
        x2t[:, 0:1, :],
        x2t[:, 1:2, :],
        x2t[:, 2:3, :],
    )
    col_spec = pl.BlockSpec((1, N, 1), lambda b: (b, 0, 0))
    row_spec = pl.BlockSpec((1, 1, N), lambda b: (b, 0, 0))
    out = pl.pallas_call(
        _emd_body,
        grid=(B,),
        in_specs=[col_spec, col_spec, col_spec, row_spec, row_spec, row_spec],
        out_specs=pl.BlockSpec((1, 1, 1), lambda b: (b, 0, 0)),
        out_shape=jax.ShapeDtypeStruct((B, 1, 1), jnp.float32),
        scratch_shapes=[
            pltpu.VMEM((N, N), _F32),
            pltpu.VMEM((N, N), _BF16),
            pltpu.VMEM((N, N), _BF16),
            pltpu.VMEM((N, N), _BF16),
            pltpu.VMEM((1, N), _F32),
            pltpu.VMEM((1, N), _F32),
            pltpu.VMEM((1, N), _BF16),
        ],
    )(*ins)
    return out[:, 0, 0]


# speedup vs baseline: 1.5361x; 1.0281x over previous
import jax
import jax.numpy as jnp
from jax.experimental import pallas as pl
from jax.experimental.pallas import tpu as pltpu

N = 2048
T = 256
NT = N // T

_F32 = jnp.float32
_BF16 = jnp.bfloat16


def _emd_body(x1a, x1b, x1c, x2a, x2b, x2c, out_ref,
              d2_ref, d_ref, e0_ref, e1_ref, rc0_ref, rc1_ref, ab_ref):
    b1 = x2a[0]
    b2 = x2b[0]
    b3 = x2c[0]

    ones_col = jnp.ones((N, 1), dtype=_BF16)

    for t in range(NT):
        rs = pl.ds(t * T, T)
        p1 = x1a[0, rs, :]
        p2 = x1b[0, rs, :]
        p3 = x1c[0, rs, :]
        d2_t = (p1 - b1) ** 2 + (p2 - b2) ** 2 + (p3 - b3) ** 2
        d2_ref[rs, :] = d2_t
        d_ref[rs, :] = jnp.sqrt(jnp.maximum(d2_t, 1e-12)).astype(_BF16)
        e_t = jnp.exp((-(4.0 ** 8)) * d2_t).astype(_BF16)
        e0_ref[rs, :] = e_t
        rc0_ref[:, rs] = jnp.transpose(
            jax.lax.dot(e_t, ones_col, preferred_element_type=_F32))

    satl = jnp.ones((1, N), dtype=_F32)
    satr = jnp.ones((1, N), dtype=_F32)
    cost = jnp.zeros((1, N), dtype=_F32)
    s = rc0_ref[:, :]

    for idx in range(10):
        j = 8 - idx
        cur = e0_ref if idx % 2 == 0 else e1_ref
        nxt = e1_ref if idx % 2 == 0 else e0_ref
        has_next = idx < 9
        level_next = -(4.0 ** (j - 1))

        a = satl / (s + 1e-9)
        ab_ref[:, :] = a.astype(_BF16)

        def pass_l1(t, cs):
            rs = pl.ds(t * T, T)
            return cs + jax.lax.dot(
                ab_ref[:, rs], cur[rs, :], preferred_element_type=_F32
            )

        cs = jax.lax.fori_loop(0, NT, pass_l1,
                               jnp.zeros((1, N), dtype=_F32), unroll=4)

        ssr = satr * cs
        r = jnp.minimum(satr / (ssr + 1e-9), 1.0)
        u = satr * r
        satr = jnp.maximum(satr - ssr * r, 0.0)
        u_b = u.astype(_BF16)
        satr_b = satr.astype(_BF16)

        def pass_l2(t, cost_c):
            rs = pl.ds(t * T, T)
            e_t = cur[rs, :]
            q_t = e_t * u_b
            r_t = q_t * d_ref[rs, :]
            rc0_ref[:, rs] = jnp.transpose(
                jax.lax.dot(q_t, ones_col, preferred_element_type=_F32))
            if has_next:
                e2_t = jnp.exp(level_next * d2_ref[rs, :]).astype(_BF16)
                nxt[rs, :] = e2_t
                p_t = e2_t * satr_b
                rc1_ref[:, rs] = jnp.transpose(
                    jax.lax.dot(p_t, ones_col, preferred_element_type=_F32))
            return cost_c + jax.lax.dot(
                ab_ref[:, rs], r_t, preferred_element_type=_F32
            )

        cost = jax.lax.fori_loop(0, NT, pass_l2, cost, unroll=2)

        eu = rc0_ref[:, :]
        satl = jnp.maximum(satl - a * eu, 0.0)
        if has_next:
            s = rc1_ref[:, :]

    s0 = jnp.sum(satr) + 1e-9
    lsum = jnp.sum(satl)
    ss = satr * (lsum / s0) + 1e-9
    r = jnp.minimum(satr / ss, 1.0)
    u_b = (satr * r).astype(_BF16)
    ab_ref[:, :] = (satl * (1.0 / s0)).astype(_BF16)

    for t in range(NT):
        rs = pl.ds(t * T, T)
        r_t = d_ref[rs, :] * u_b
        cost = cost + jax.lax.dot(
            ab_ref[:, rs], r_t, preferred_element_type=_F32
        )

    out_ref[0] = jnp.sum(cost, axis=1, keepdims=True)


def kernel(input1, input2):
    B = input1.shape[0]
    x2t = jnp.transpose(input2, (0, 2, 1))
    ins = (
        input1[:, :, 0:1],
        input1[:, :, 1:2],
        input1[:, :, 2:3],
        x2t[:, 0:1, :],
        x2t[:, 1:2, :],
        x2t[:, 2:3, :],
    )
    col_spec = pl.BlockSpec((1, N, 1), lambda b: (b, 0, 0))
    row_spec = pl.BlockSpec((1, 1, N), lambda b: (b, 0, 0))
    out = pl.pallas_call(
        _emd_body,
        grid=(B,),
        in_specs=[col_spec, col_spec, col_spec, row_spec, row_spec, row_spec],
        out_specs=pl.BlockSpec((1, 1, 1), lambda b: (b, 0, 0)),
        out_shape=jax.ShapeDtypeStruct((B, 1, 1), jnp.float32),
        scratch_shapes=[
            pltpu.VMEM((N, N), _F32),
            pltpu.VMEM((N, N), _BF16),
            pltpu.VMEM((N, N), _BF16),
            pltpu.VMEM((N, N), _BF16),
            pltpu.VMEM((1, N), _F32),
            pltpu.VMEM((1, N), _F32),
            pltpu.VMEM((1, N), _BF16),
        ],
    )(*ins)
    return out[:, 0, 0]
